# baseline (speedup 1.0000x reference)
import functools

import jax
import jax.numpy as jnp
from jax import lax
from jax.experimental import pallas as pl
from jax.experimental.pallas import tpu as pltpu
from jax.experimental.pallas import tpu_sc as plsc

T = 2048
H = 1024
E = 8
I = 512
K = 2
B = 256
R = T * K
NB = R // B + E - 1
NBB = NB * B

NC, NS = 2, 16
NW = NC * NS
_ROWS_PER_W = R // NW
_CH = 64


def _router_body(x_ref, gw_ref, pos_ref, wtf_ref, be_ref):
    x = x_ref[...]
    logits = lax.dot_general(x, gw_ref[...], (((1,), (1,)), ((), ())),
                             preferred_element_type=jnp.float32)
    col = lax.broadcasted_iota(jnp.int32, (T, E), 1)
    m1 = jnp.max(logits, axis=1, keepdims=True)
    a1 = jnp.min(jnp.where(logits == m1, col, E), axis=1, keepdims=True)
    masked = jnp.where(col == a1, -jnp.float32(3.0e38), logits)
    m2 = jnp.max(masked, axis=1, keepdims=True)
    a2 = jnp.min(jnp.where(masked == m2, col, E), axis=1, keepdims=True)
    wa = jax.nn.sigmoid(m1 - m2)
    wb = 1.0 - wa

    oh1 = (col == a1).astype(jnp.float32)
    oh2 = (col == a2).astype(jnp.float32)
    M = jnp.concatenate([oh1, oh2], axis=0)

    S = M
    d = 1
    while d < R:
        shifted = jnp.concatenate(
            [jnp.zeros((d, E), jnp.float32), S[: R - d, :]], axis=0)
        S = S + shifted
        d *= 2

    rank = jnp.sum(M * S, axis=1, keepdims=True) - 1.0
    counts = S[R - 1: R, :]
    nb = (counts.astype(jnp.int32) + (B - 1)) // B
    r_i = lax.broadcasted_iota(jnp.int32, (E, E), 0)
    c_i = lax.broadcasted_iota(jnp.int32, (E, E), 1)
    tri8 = (r_i <= c_i).astype(jnp.float32)
    cumnb = lax.dot_general(nb.astype(jnp.float32), tri8,
                            (((1,), (0,)), ((), ())),
                            preferred_element_type=jnp.float32)
    poff = (B * (cumnb - nb.astype(jnp.float32)))
    posf = jnp.sum(M * poff, axis=1, keepdims=True) + rank
    pos_ref[...] = posf.astype(jnp.int32)
    wtf_ref[...] = jnp.concatenate([wa, wb], axis=0)

    eye8 = (r_i == c_i).astype(jnp.float32)
    cumnb_col = lax.dot_general(eye8, cumnb, (((1,), (1,)), ((), ())),
                                preferred_element_type=jnp.float32)
    bi = lax.broadcasted_iota(jnp.int32, (E, 128), 1)
    s2 = (bi.astype(jnp.float32) >= cumnb_col).astype(jnp.float32)
    ber = lax.dot_general(jnp.ones((1, E), jnp.float32), s2,
                          (((1,), (0,)), ((), ())),
                          preferred_element_type=jnp.float32)
    be_ref[...] = jnp.minimum(ber, E - 1).astype(jnp.int32)


def _router_call(hidden, gate_weight):
    return pl.pallas_call(
        _router_body,
        in_specs=[
            pl.BlockSpec((T, H), lambda: (0, 0)),
            pl.BlockSpec((E, H), lambda: (0, 0)),
        ],
        out_specs=[
            pl.BlockSpec((R, 1), lambda: (0, 0)),
            pl.BlockSpec((R, 1), lambda: (0, 0)),
            pl.BlockSpec((1, 128), lambda: (0, 0)),
        ],
        out_shape=[
            jax.ShapeDtypeStruct((R, 1), jnp.int32),
            jax.ShapeDtypeStruct((R, 1), jnp.float32),
            jax.ShapeDtypeStruct((1, 128), jnp.int32),
        ],
    )(hidden, gate_weight)


@functools.cache
def _make_scatter():
    mesh = plsc.VectorSubcoreMesh(core_axis_name="c", subcore_axis_name="s",
                                  num_cores=NC, num_subcores=NS)

    @functools.partial(
        pl.kernel, mesh=mesh,
        out_type=jax.ShapeDtypeStruct((NBB, H), jnp.float32),
        scratch_types=[
            pltpu.VMEM((_CH,), jnp.int32),
            pltpu.VMEM((_CH,), jnp.int32),
            pltpu.VMEM((_CH, H), jnp.float32),
            pltpu.SemaphoreType.DMA,
        ],
    )
    def scatter_k(hidden_hbm, pos_hbm, xs_hbm, idx0, idx1, rows, sem):
        wid = lax.axis_index("s") * NC + lax.axis_index("c")
        r0 = wid * _ROWS_PER_W
        idx_bufs = (idx0, idx1)
        for chunk in range(_ROWS_PER_W // _CH):
            rbase = r0 + chunk * _CH
            tbase = lax.rem(rbase, T)
            idx = idx_bufs[chunk]
            pltpu.sync_copy(pos_hbm.at[pl.ds(rbase, _CH)], idx)
            pltpu.sync_copy(hidden_hbm.at[pl.ds(tbase, _CH)], rows)
            pltpu.async_copy(rows, xs_hbm.at[idx], sem).wait()

    return scatter_k


def _ffn_body(be_ref, x_ref, w13_ref, w2_ref, y_ref):
    x = x_ref[...].astype(jnp.bfloat16)
    h = lax.dot_general(x, w13_ref[0], (((1,), (1,)), ((), ())),
                        preferred_element_type=jnp.float32)
    gate = h[:, :I]
    up = h[:, I:]
    act = (gate * jax.nn.sigmoid(gate) * up).astype(jnp.bfloat16)
    y_ref[...] = lax.dot_general(act, w2_ref[0], (((1,), (1,)), ((), ())),
                                 preferred_element_type=jnp.float32)


def _ffn_call(be, x_sorted, w13, w2):
    grid_spec = pltpu.PrefetchScalarGridSpec(
        num_scalar_prefetch=1,
        grid=(NB,),
        in_specs=[
            pl.BlockSpec((B, H), lambda b, be_ref: (b, 0)),
            pl.BlockSpec((1, 2 * I, H), lambda b, be_ref: (be_ref[b], 0, 0)),
            pl.BlockSpec((1, H, I), lambda b, be_ref: (be_ref[b], 0, 0)),
        ],
        out_specs=pl.BlockSpec((B, H), lambda b, be_ref: (b, 0)),
    )
    return pl.pallas_call(
        _ffn_body,
        grid_spec=grid_spec,
        out_shape=jax.ShapeDtypeStruct((NBB, H), jnp.float32),
    )(be, x_sorted, w13, w2)


_TOK_PER_W = T // NW
_TCH = 32


@functools.cache
def _make_combine():
    mesh = plsc.VectorSubcoreMesh(core_axis_name="c", subcore_axis_name="s",
                                  num_cores=NC, num_subcores=NS)

    @functools.partial(
        pl.kernel, mesh=mesh,
        out_type=jax.ShapeDtypeStruct((T, H), jnp.float32),
        scratch_types=[
            pltpu.VMEM((_TCH,), jnp.int32),
            pltpu.VMEM((_TCH,), jnp.int32),
            pltpu.VMEM((_TOK_PER_W,), jnp.float32),
            pltpu.VMEM((_TOK_PER_W,), jnp.float32),
            pltpu.VMEM((_TCH, H), jnp.float32),
            pltpu.VMEM((_TCH, H), jnp.float32),
            pltpu.VMEM((_TCH, H), jnp.float32),
            pltpu.SemaphoreType.DMA,
        ],
    )
    def combine_k(y_hbm, pos_hbm, wtf_hbm, out_hbm,
                  idx0, idx1, w0v, w1v, rows0, rows1, outb, sem):
        wid = lax.axis_index("s") * NC + lax.axis_index("c")
        tb = wid * _TOK_PER_W
        pltpu.sync_copy(wtf_hbm.at[pl.ds(tb, _TOK_PER_W)], w0v)
        pltpu.sync_copy(wtf_hbm.at[pl.ds(T + tb, _TOK_PER_W)], w1v)
        for chunk in range(_TOK_PER_W // _TCH):
            cb = tb + chunk * _TCH
            pltpu.sync_copy(pos_hbm.at[pl.ds(cb, _TCH)], idx0)
            pltpu.sync_copy(pos_hbm.at[pl.ds(T + cb, _TCH)], idx1)
            pltpu.async_copy(y_hbm.at[idx0], rows0, sem).wait()
            pltpu.async_copy(y_hbm.at[idx1], rows1, sem).wait()

            for grp in range(_TCH // 16):
                w0g = w0v[pl.ds(chunk * _TCH + grp * 16, 16)]
                w1g = w1v[pl.ds(chunk * _TCH + grp * 16, 16)]

                def tok_body(j, _):
                    jv = jnp.full((16,), j, jnp.int32)
                    w0 = w0g.at[jv].get(mode="promise_in_bounds")
                    w1 = w1g.at[jv].get(mode="promise_in_bounds")
                    jr = grp * 16 + j

                    def col_body(c, _):
                        sl = pl.ds(c * 16, 16)
                        outb[jr, sl] = w0 * rows0[jr, sl] + w1 * rows1[jr, sl]
                        return 0

                    return lax.fori_loop(0, H // 16, col_body, 0)

                lax.fori_loop(0, 16, tok_body, 0)
            pltpu.sync_copy(outb, out_hbm.at[pl.ds(cb, _TCH)])

    return combine_k


def kernel(hidden_states, gate_weight, w13, w2):
    pos, wtf, be = _router_call(hidden_states, gate_weight)
    pos_flat = pos.reshape(R)
    wtf_flat = wtf.reshape(R)
    be_nb = be.reshape(128)[:NB]
    x_sorted = _make_scatter()(hidden_states, pos_flat)
    y = _ffn_call(be_nb, x_sorted,
                  w13.astype(jnp.bfloat16), w2.astype(jnp.bfloat16))
    return _make_combine()(y, pos_flat, wtf_flat)


# speedup vs baseline: 1.0132x; 1.0132x over previous
import functools

import jax
import jax.numpy as jnp
from jax import lax
from jax.experimental import pallas as pl
from jax.experimental.pallas import tpu as pltpu
from jax.experimental.pallas import tpu_sc as plsc

T = 2048
H = 1024
E = 8
I = 512
K = 2
B = 256
R = T * K
NB = R // B + E - 1
NBB = NB * B

NC, NS = 2, 16
NW = NC * NS
_ROWS_PER_W = R // NW
_CH = 64


def _router_body(x_ref, gw_ref, pos_ref, wtf_ref, be_ref):
    x = x_ref[...]
    logits = lax.dot_general(x, gw_ref[...], (((1,), (1,)), ((), ())),
                             preferred_element_type=jnp.float32)
    col = lax.broadcasted_iota(jnp.int32, (T, E), 1)
    m1 = jnp.max(logits, axis=1, keepdims=True)
    a1 = jnp.min(jnp.where(logits == m1, col, E), axis=1, keepdims=True)
    masked = jnp.where(col == a1, -jnp.float32(3.0e38), logits)
    m2 = jnp.max(masked, axis=1, keepdims=True)
    a2 = jnp.min(jnp.where(masked == m2, col, E), axis=1, keepdims=True)
    wa = jax.nn.sigmoid(m1 - m2)
    wb = 1.0 - wa

    oh1 = (col == a1).astype(jnp.float32)
    oh2 = (col == a2).astype(jnp.float32)
    M = jnp.concatenate([oh1, oh2], axis=0)

    S = M
    d = 1
    while d < R:
        shifted = jnp.concatenate(
            [jnp.zeros((d, E), jnp.float32), S[: R - d, :]], axis=0)
        S = S + shifted
        d *= 2

    rank = jnp.sum(M * S, axis=1, keepdims=True) - 1.0
    counts = S[R - 1: R, :]
    nb = (counts.astype(jnp.int32) + (B - 1)) // B
    r_i = lax.broadcasted_iota(jnp.int32, (E, E), 0)
    c_i = lax.broadcasted_iota(jnp.int32, (E, E), 1)
    tri8 = (r_i <= c_i).astype(jnp.float32)
    cumnb = lax.dot_general(nb.astype(jnp.float32), tri8,
                            (((1,), (0,)), ((), ())),
                            preferred_element_type=jnp.float32)
    poff = (B * (cumnb - nb.astype(jnp.float32)))
    posf = jnp.sum(M * poff, axis=1, keepdims=True) + rank
    pos_ref[...] = posf.astype(jnp.int32)
    wtf_ref[...] = jnp.concatenate([wa, wb], axis=0)

    eye8 = (r_i == c_i).astype(jnp.float32)
    cumnb_col = lax.dot_general(eye8, cumnb, (((1,), (1,)), ((), ())),
                                preferred_element_type=jnp.float32)
    bi = lax.broadcasted_iota(jnp.int32, (E, 128), 1)
    s2 = (bi.astype(jnp.float32) >= cumnb_col).astype(jnp.float32)
    ber = lax.dot_general(jnp.ones((1, E), jnp.float32), s2,
                          (((1,), (0,)), ((), ())),
                          preferred_element_type=jnp.float32)
    be_ref[...] = jnp.minimum(ber, E - 1).astype(jnp.int32)


def _router_call(hidden, gate_weight):
    return pl.pallas_call(
        _router_body,
        in_specs=[
            pl.BlockSpec((T, H), lambda: (0, 0)),
            pl.BlockSpec((E, H), lambda: (0, 0)),
        ],
        out_specs=[
            pl.BlockSpec((R, 1), lambda: (0, 0)),
            pl.BlockSpec((R, 1), lambda: (0, 0)),
            pl.BlockSpec((1, 128), lambda: (0, 0)),
        ],
        out_shape=[
            jax.ShapeDtypeStruct((R, 1), jnp.int32),
            jax.ShapeDtypeStruct((R, 1), jnp.float32),
            jax.ShapeDtypeStruct((1, 128), jnp.int32),
        ],
    )(hidden, gate_weight)


@functools.cache
def _make_scatter():
    mesh = plsc.VectorSubcoreMesh(core_axis_name="c", subcore_axis_name="s",
                                  num_cores=NC, num_subcores=NS)

    @functools.partial(
        pl.kernel, mesh=mesh,
        out_type=jax.ShapeDtypeStruct((NBB, H), jnp.float32),
        scratch_types=[
            pltpu.VMEM((_CH,), jnp.int32),
            pltpu.VMEM((_CH,), jnp.int32),
            pltpu.VMEM((_CH, H), jnp.float32),
            pltpu.SemaphoreType.DMA,
        ],
    )
    def scatter_k(hidden_hbm, pos_hbm, xs_hbm, idx0, idx1, rows, sem):
        wid = lax.axis_index("s") * NC + lax.axis_index("c")
        r0 = wid * _ROWS_PER_W
        idx_bufs = (idx0, idx1)
        for chunk in range(_ROWS_PER_W // _CH):
            rbase = r0 + chunk * _CH
            tbase = lax.rem(rbase, T)
            idx = idx_bufs[chunk]
            pltpu.sync_copy(pos_hbm.at[pl.ds(rbase, _CH)], idx)
            pltpu.sync_copy(hidden_hbm.at[pl.ds(tbase, _CH)], rows)
            pltpu.async_copy(rows, xs_hbm.at[idx], sem).wait()

    return scatter_k


def _ffn_body(be_ref, x_ref, w13_ref, w2_ref, y_ref):
    x = x_ref[...]
    h = lax.dot_general(x, w13_ref[0], (((1,), (1,)), ((), ())),
                        preferred_element_type=jnp.float32)
    gate = h[:, :I]
    up = h[:, I:]
    act = gate * jax.nn.sigmoid(gate) * up
    y_ref[...] = lax.dot_general(act, w2_ref[0], (((1,), (1,)), ((), ())),
                                 preferred_element_type=jnp.float32)


def _ffn_call(be, x_sorted, w13, w2):
    grid_spec = pltpu.PrefetchScalarGridSpec(
        num_scalar_prefetch=1,
        grid=(NB,),
        in_specs=[
            pl.BlockSpec((B, H), lambda b, be_ref: (b, 0)),
            pl.BlockSpec((1, 2 * I, H), lambda b, be_ref: (be_ref[b], 0, 0)),
            pl.BlockSpec((1, H, I), lambda b, be_ref: (be_ref[b], 0, 0)),
        ],
        out_specs=pl.BlockSpec((B, H), lambda b, be_ref: (b, 0)),
    )
    return pl.pallas_call(
        _ffn_body,
        grid_spec=grid_spec,
        out_shape=jax.ShapeDtypeStruct((NBB, H), jnp.float32),
    )(be, x_sorted, w13, w2)


_TOK_PER_W = T // NW
_TCH = 32


@functools.cache
def _make_combine():
    mesh = plsc.VectorSubcoreMesh(core_axis_name="c", subcore_axis_name="s",
                                  num_cores=NC, num_subcores=NS)

    @functools.partial(
        pl.kernel, mesh=mesh,
        out_type=jax.ShapeDtypeStruct((T, H), jnp.float32),
        scratch_types=[
            pltpu.VMEM((_TCH,), jnp.int32),
            pltpu.VMEM((_TCH,), jnp.int32),
            pltpu.VMEM((_TOK_PER_W,), jnp.float32),
            pltpu.VMEM((_TOK_PER_W,), jnp.float32),
            pltpu.VMEM((_TCH, H), jnp.float32),
            pltpu.VMEM((_TCH, H), jnp.float32),
            pltpu.VMEM((_TCH, H), jnp.float32),
            pltpu.SemaphoreType.DMA,
        ],
    )
    def combine_k(y_hbm, pos_hbm, wtf_hbm, out_hbm,
                  idx0, idx1, w0v, w1v, rows0, rows1, outb, sem):
        wid = lax.axis_index("s") * NC + lax.axis_index("c")
        tb = wid * _TOK_PER_W
        pltpu.sync_copy(wtf_hbm.at[pl.ds(tb, _TOK_PER_W)], w0v)
        pltpu.sync_copy(wtf_hbm.at[pl.ds(T + tb, _TOK_PER_W)], w1v)
        for chunk in range(_TOK_PER_W // _TCH):
            cb = tb + chunk * _TCH
            pltpu.sync_copy(pos_hbm.at[pl.ds(cb, _TCH)], idx0)
            pltpu.sync_copy(pos_hbm.at[pl.ds(T + cb, _TCH)], idx1)
            pltpu.async_copy(y_hbm.at[idx0], rows0, sem).wait()
            pltpu.async_copy(y_hbm.at[idx1], rows1, sem).wait()

            for grp in range(_TCH // 16):
                w0g = w0v[pl.ds(chunk * _TCH + grp * 16, 16)]
                w1g = w1v[pl.ds(chunk * _TCH + grp * 16, 16)]

                def tok_body(j, _):
                    jv = jnp.full((16,), j, jnp.int32)
                    w0 = w0g.at[jv].get(mode="promise_in_bounds")
                    w1 = w1g.at[jv].get(mode="promise_in_bounds")
                    jr = grp * 16 + j

                    def col_body(c, _):
                        sl = pl.ds(c * 16, 16)
                        outb[jr, sl] = w0 * rows0[jr, sl] + w1 * rows1[jr, sl]
                        return 0

                    return lax.fori_loop(0, H // 16, col_body, 0, unroll=8)

                lax.fori_loop(0, 16, tok_body, 0)
            pltpu.sync_copy(outb, out_hbm.at[pl.ds(cb, _TCH)])

    return combine_k


def kernel(hidden_states, gate_weight, w13, w2):
    pos, wtf, be = _router_call(hidden_states, gate_weight)
    pos_flat = pos.reshape(R)
    wtf_flat = wtf.reshape(R)
    be_nb = be.reshape(128)[:NB]
    x_sorted = _make_scatter()(hidden_states, pos_flat)
    y = _ffn_call(be_nb, x_sorted, w13, w2)
    return _make_combine()(y, pos_flat, wtf_flat)


# speedup vs baseline: 1.2667x; 1.2502x over previous
import functools

import jax
import jax.numpy as jnp
from jax import lax
from jax.experimental import pallas as pl
from jax.experimental.pallas import tpu as pltpu
from jax.experimental.pallas import tpu_sc as plsc

T = 2048
H = 1024
E = 8
I = 512
K = 2
B = 256
R = T * K
NB = R // B + E - 1
NBB = NB * B

NC, NS = 2, 16
NW = NC * NS
_ROWS_PER_W = R // NW
_CH = 64


def _router_body(x_ref, gw_ref, pos_ref, wtf_ref, be_ref):
    x = x_ref[...]
    logits = lax.dot_general(x, gw_ref[...], (((1,), (1,)), ((), ())),
                             preferred_element_type=jnp.float32)
    col = lax.broadcasted_iota(jnp.int32, (T, E), 1)
    m1 = jnp.max(logits, axis=1, keepdims=True)
    a1 = jnp.min(jnp.where(logits == m1, col, E), axis=1, keepdims=True)
    masked = jnp.where(col == a1, -jnp.float32(3.0e38), logits)
    m2 = jnp.max(masked, axis=1, keepdims=True)
    a2 = jnp.min(jnp.where(masked == m2, col, E), axis=1, keepdims=True)
    wa = jax.nn.sigmoid(m1 - m2)
    wb = 1.0 - wa

    oh1 = (col == a1).astype(jnp.float32)
    oh2 = (col == a2).astype(jnp.float32)
    M = jnp.concatenate([oh1, oh2], axis=0)

    S = M
    d = 1
    while d < R:
        shifted = jnp.concatenate(
            [jnp.zeros((d, E), jnp.float32), S[: R - d, :]], axis=0)
        S = S + shifted
        d *= 2

    rank = jnp.sum(M * S, axis=1, keepdims=True) - 1.0
    counts = S[R - 1: R, :]
    nb = (counts.astype(jnp.int32) + (B - 1)) // B
    r_i = lax.broadcasted_iota(jnp.int32, (E, E), 0)
    c_i = lax.broadcasted_iota(jnp.int32, (E, E), 1)
    tri8 = (r_i <= c_i).astype(jnp.float32)
    cumnb = lax.dot_general(nb.astype(jnp.float32), tri8,
                            (((1,), (0,)), ((), ())),
                            preferred_element_type=jnp.float32)
    poff = (B * (cumnb - nb.astype(jnp.float32)))
    posf = jnp.sum(M * poff, axis=1, keepdims=True) + rank
    pos_ref[...] = posf.astype(jnp.int32)
    wtf_ref[...] = jnp.concatenate([wa, wb], axis=0)

    eye8 = (r_i == c_i).astype(jnp.float32)
    cumnb_col = lax.dot_general(eye8, cumnb, (((1,), (1,)), ((), ())),
                                preferred_element_type=jnp.float32)
    bi = lax.broadcasted_iota(jnp.int32, (E, 128), 1)
    s2 = (bi.astype(jnp.float32) >= cumnb_col).astype(jnp.float32)
    ber = lax.dot_general(jnp.ones((1, E), jnp.float32), s2,
                          (((1,), (0,)), ((), ())),
                          preferred_element_type=jnp.float32)
    be_ref[...] = jnp.minimum(ber, E - 1).astype(jnp.int32)


def _router_call(hidden, gate_weight):
    return pl.pallas_call(
        _router_body,
        in_specs=[
            pl.BlockSpec((T, H), lambda: (0, 0)),
            pl.BlockSpec((E, H), lambda: (0, 0)),
        ],
        out_specs=[
            pl.BlockSpec((R, 1), lambda: (0, 0)),
            pl.BlockSpec((R, 1), lambda: (0, 0)),
            pl.BlockSpec((1, 128), lambda: (0, 0)),
        ],
        out_shape=[
            jax.ShapeDtypeStruct((R, 1), jnp.int32),
            jax.ShapeDtypeStruct((R, 1), jnp.float32),
            jax.ShapeDtypeStruct((1, 128), jnp.int32),
        ],
    )(hidden, gate_weight)


@functools.cache
def _make_scatter():
    mesh = plsc.VectorSubcoreMesh(core_axis_name="c", subcore_axis_name="s",
                                  num_cores=NC, num_subcores=NS)

    @functools.partial(
        pl.kernel, mesh=mesh,
        out_type=jax.ShapeDtypeStruct((NBB, H), jnp.float32),
        scratch_types=[
            pltpu.VMEM((_CH,), jnp.int32),
            pltpu.VMEM((_CH,), jnp.int32),
            pltpu.VMEM((_CH, H), jnp.float32),
            pltpu.SemaphoreType.DMA,
        ],
    )
    def scatter_k(hidden_hbm, pos_hbm, xs_hbm, idx0, idx1, rows, sem):
        wid = lax.axis_index("s") * NC + lax.axis_index("c")
        r0 = wid * _ROWS_PER_W
        idx_bufs = (idx0, idx1)
        for chunk in range(_ROWS_PER_W // _CH):
            rbase = r0 + chunk * _CH
            tbase = lax.rem(rbase, T)
            idx = idx_bufs[chunk]
            pltpu.sync_copy(pos_hbm.at[pl.ds(rbase, _CH)], idx)
            pltpu.sync_copy(hidden_hbm.at[pl.ds(tbase, _CH)], rows)
            pltpu.async_copy(rows, xs_hbm.at[idx], sem).wait()

    return scatter_k


def _ffn_body(be_ref, x_ref, w13_ref, w2_ref, y_ref):
    x = x_ref[...]
    h = lax.dot_general(x, w13_ref[0], (((1,), (1,)), ((), ())),
                        preferred_element_type=jnp.float32)
    gate = h[:, :I]
    up = h[:, I:]
    act = gate * jax.nn.sigmoid(gate) * up
    y_ref[...] = lax.dot_general(act, w2_ref[0], (((1,), (1,)), ((), ())),
                                 preferred_element_type=jnp.float32)


def _ffn_call(be, x_sorted, w13, w2):
    grid_spec = pltpu.PrefetchScalarGridSpec(
        num_scalar_prefetch=1,
        grid=(NB,),
        in_specs=[
            pl.BlockSpec((B, H), lambda b, be_ref: (b, 0)),
            pl.BlockSpec((1, 2 * I, H), lambda b, be_ref: (be_ref[b], 0, 0)),
            pl.BlockSpec((1, H, I), lambda b, be_ref: (be_ref[b], 0, 0)),
        ],
        out_specs=pl.BlockSpec((B, H), lambda b, be_ref: (b, 0)),
    )
    return pl.pallas_call(
        _ffn_body,
        grid_spec=grid_spec,
        out_shape=jax.ShapeDtypeStruct((NBB, H), jnp.float32),
    )(be, x_sorted, w13, w2)


_TOK_PER_W = T // NW
_TCH = 16
_NCH = _TOK_PER_W // _TCH
_RING = 2


@functools.cache
def _make_combine():
    mesh = plsc.VectorSubcoreMesh(core_axis_name="c", subcore_axis_name="s",
                                  num_cores=NC, num_subcores=NS)

    @functools.partial(
        pl.kernel, mesh=mesh,
        out_type=jax.ShapeDtypeStruct((T, H), jnp.float32),
        scratch_types=(
            [pltpu.VMEM((_TCH,), jnp.int32) for _ in range(2 * _RING)]
            + [pltpu.VMEM((_TOK_PER_W,), jnp.float32) for _ in range(2)]
            + [pltpu.VMEM((_TCH, H), jnp.float32) for _ in range(2 * _RING)]
            + [pltpu.VMEM((_TCH, H), jnp.float32) for _ in range(_RING)]
            + [pltpu.SemaphoreType.DMA for _ in range(2 * _RING + _RING)]
        ),
    )
    def combine_k(y_hbm, pos_hbm, wtf_hbm, out_hbm,
                  i00, i10, i01, i11, w0v, w1v,
                  r00, r10, r01, r11, ob0, ob1,
                  sg00, sg10, sg01, sg11, sw0, sw1):
        idx = ((i00, i10), (i01, i11))
        rows = ((r00, r10), (r01, r11))
        outb = (ob0, ob1)
        gsem = ((sg00, sg10), (sg01, sg11))
        wsem = (sw0, sw1)

        wid = lax.axis_index("s") * NC + lax.axis_index("c")
        tb = wid * _TOK_PER_W
        pltpu.sync_copy(wtf_hbm.at[pl.ds(tb, _TOK_PER_W)], w0v)
        pltpu.sync_copy(wtf_hbm.at[pl.ds(T + tb, _TOK_PER_W)], w1v)

        def issue(c, slot):
            cb = tb + c * _TCH
            pltpu.sync_copy(pos_hbm.at[pl.ds(cb, _TCH)], idx[slot][0])
            pltpu.sync_copy(pos_hbm.at[pl.ds(T + cb, _TCH)], idx[slot][1])
            g0 = pltpu.async_copy(y_hbm.at[idx[slot][0]], rows[slot][0],
                                  gsem[slot][0])
            g1 = pltpu.async_copy(y_hbm.at[idx[slot][1]], rows[slot][1],
                                  gsem[slot][1])
            return (g0, g1)

        gathers = [None] * _NCH
        writes = [None] * _NCH
        gathers[0] = issue(0, 0)
        for c in range(_NCH):
            slot = c % _RING
            if c + 1 < _NCH:
                gathers[c + 1] = issue(c + 1, (c + 1) % _RING)
            gathers[c][0].wait()
            gathers[c][1].wait()
            if c >= _RING:
                writes[c - _RING].wait()
            w0g = w0v[pl.ds(c * _TCH, 16)]
            w1g = w1v[pl.ds(c * _TCH, 16)]
            r0, r1, ob = rows[slot][0], rows[slot][1], outb[slot]

            def tok_body(j, _):
                jv = jnp.full((16,), j, jnp.int32)
                w0 = w0g.at[jv].get(mode="promise_in_bounds")
                w1 = w1g.at[jv].get(mode="promise_in_bounds")

                def col_body(cc, _):
                    sl = pl.ds(cc * 16, 16)
                    ob[j, sl] = w0 * r0[j, sl] + w1 * r1[j, sl]
                    return 0

                return lax.fori_loop(0, H // 16, col_body, 0, unroll=4)

            lax.fori_loop(0, _TCH, tok_body, 0)
            writes[c] = pltpu.async_copy(
                ob, out_hbm.at[pl.ds(tb + c * _TCH, _TCH)], wsem[slot])
        for c in range(_NCH - _RING, _NCH):
            writes[c].wait()

    return combine_k


def kernel(hidden_states, gate_weight, w13, w2):
    pos, wtf, be = _router_call(hidden_states, gate_weight)
    pos_flat = pos.reshape(R)
    wtf_flat = wtf.reshape(R)
    be_nb = be.reshape(128)[:NB]
    x_sorted = _make_scatter()(hidden_states, pos_flat)
    y = _ffn_call(be_nb, x_sorted, w13, w2)
    return _make_combine()(y, pos_flat, wtf_flat)
